# baseline (speedup 1.0000x reference)
import functools

import jax
import jax.numpy as jnp
from jax import lax
from jax.experimental import pallas as pl
from jax.experimental.pallas import tpu as pltpu
from jax.experimental.pallas import tpu_sc as plsc

_LANES = 16
_NUM_WORKERS = 32
_CHUNK = 128


def _build_sc_call(n_rows, seq_len, d_model):
    rpw = n_rows // _NUM_WORKERS
    n_chunks = rpw // _CHUNK
    n_vec = d_model // _LANES
    mesh = plsc.VectorSubcoreMesh(
        core_axis_name="c", subcore_axis_name="s", num_cores=2, num_subcores=16
    )

    @functools.partial(
        pl.kernel,
        out_type=jax.ShapeDtypeStruct((n_rows, d_model), jnp.float32),
        mesh=mesh,
        scratch_types=[
            pltpu.VMEM((rpw,), jnp.int32),
            pltpu.VMEM((2, _CHUNK), jnp.int32),
            pltpu.VMEM((2 * seq_len, d_model), jnp.float32),
            pltpu.VMEM((2, _CHUNK, 2 * d_model), jnp.float32),
            pltpu.VMEM((2, _CHUNK, d_model), jnp.float32),
            pltpu.SemaphoreType.DMA,
            pltpu.SemaphoreType.DMA,
            pltpu.SemaphoreType.DMA,
            pltpu.SemaphoreType.DMA,
        ],
        compiler_params=pltpu.CompilerParams(needs_layout_passes=False),
    )
    def sc_encode(idx_hbm, tab_hbm, pos2_hbm, out_hbm,
                  idx_v, sh_v, pos_v, bufs, outbufs,
                  gsem0, gsem1, osem0, osem1):
        gsems = (gsem0, gsem1)
        osems = (osem0, osem1)
        wid = lax.axis_index("s") * 2 + lax.axis_index("c")
        base = pl.multiple_of(wid * rpw, rpw)
        pltpu.sync_copy(idx_hbm.at[pl.ds(base, rpw)], idx_v)
        pltpu.sync_copy(pos2_hbm, pos_v)

        lanes_iota = lax.iota(jnp.int32, _LANES)
        _gd = lax.GatherDimensionNumbers(
            offset_dims=(), collapsed_slice_dims=(0,), start_index_map=(0,)
        )

        def gather_start(c, b):
            off = pl.multiple_of(c * _CHUNK, _CHUNK)

            def sh_body(i, carry):
                src = idx_v[pl.ds(off + i * _LANES, _LANES)]
                sh_v[b, pl.ds(i * _LANES, _LANES)] = lax.shift_right_logical(src, 1)
                return carry

            lax.fori_loop(0, _CHUNK // _LANES, sh_body, 0)
            pltpu.async_copy(tab_hbm.at[sh_v.at[b]], bufs.at[b], gsems[b])

        def gather_wait(c, b):
            pltpu.make_async_copy(
                tab_hbm.at[sh_v.at[b]], bufs.at[b], gsems[b]
            ).wait()

        def out_rows(c):
            off = pl.multiple_of(c * _CHUNK, _CHUNK)
            return out_hbm.at[pl.ds(base + off, _CHUNK)]

        def process(c, b):
            p0 = lax.rem(c * _CHUNK, seq_len)
            off = pl.multiple_of(c * _CHUNK, _CHUNK)
            buf = bufs.at[b]
            outb = outbufs.at[b]

            def row_body(r, carry):
                vec = idx_v[pl.ds(off + (r & ~(_LANES - 1)), _LANES)]
                lane = jnp.broadcast_to(r & (_LANES - 1), (_LANES,))
                par = lax.gather(
                    vec, lane[:, None], _gd, slice_sizes=(1,),
                    mode=lax.GatherScatterMode.PROMISE_IN_BOUNDS,
                )
                cols = (par & 1) * d_model + lanes_iota
                rows16 = jnp.broadcast_to(r, (_LANES,))
                pr = p0 + r
                for k in range(n_vec):
                    vals = plsc.load_gather(buf, [rows16, cols + (k * _LANES)])
                    sl = pl.ds(k * _LANES, _LANES)
                    outb[r, sl] = vals + pos_v[pr, sl]
                return carry

            lax.fori_loop(0, _CHUNK, row_body, 0)
            pltpu.async_copy(outb, out_rows(c), osems[b])

        def store_wait(c, b):
            pltpu.make_async_copy(outbufs.at[b], out_rows(c), osems[b]).wait()

        gather_start(0, 0)
        gather_start(1, 1)

        def chunk_pair(j, carry):
            for b in range(2):
                c = 2 * j + b
                gather_wait(c, b)
                @pl.when(j > 0)
                def _():
                    store_wait(c - 2, b)
                process(c, b)
                gather_start(lax.rem(c + 2, n_chunks), b)
            return carry

        lax.fori_loop(0, n_chunks // 2, chunk_pair, 0)
        store_wait(n_chunks - 2, 0)
        store_wait(n_chunks - 1, 1)
        gather_wait(0, 0)
        gather_wait(1, 1)

    return sc_encode


def kernel(inputs, emb_table, pos_table):
    batch, seq_len = inputs.shape
    n_vocab, d_model = emb_table.shape
    n_rows = batch * seq_len
    idx_flat = inputs.reshape(n_rows)
    tab128 = emb_table.reshape(n_vocab // 2, 2 * d_model)
    pos2 = jnp.concatenate([pos_table, pos_table], axis=0)
    out = _build_sc_call(n_rows, seq_len, d_model)(idx_flat, tab128, pos2)
    return out.reshape(batch, seq_len, d_model)


# speedup vs baseline: 1.0456x; 1.0456x over previous
import functools

import jax
import jax.numpy as jnp
from jax import lax
from jax.experimental import pallas as pl
from jax.experimental.pallas import tpu as pltpu
from jax.experimental.pallas import tpu_sc as plsc

_LANES = 16
_NUM_WORKERS = 32
_CHUNK = 128
_UNROLL = 4


def _build_sc_call(n_rows, seq_len, d_model):
    rpw = n_rows // _NUM_WORKERS
    n_chunks = rpw // _CHUNK
    n_vec = d_model // _LANES
    mesh = plsc.VectorSubcoreMesh(
        core_axis_name="c", subcore_axis_name="s", num_cores=2, num_subcores=16
    )

    @functools.partial(
        pl.kernel,
        out_type=jax.ShapeDtypeStruct((n_rows, d_model), jnp.float32),
        mesh=mesh,
        scratch_types=[
            pltpu.VMEM((rpw,), jnp.int32),
            pltpu.VMEM((2, _CHUNK, 2 * d_model), jnp.float32),
            pltpu.VMEM((2, _CHUNK, d_model), jnp.float32),
            pltpu.SemaphoreType.DMA,
            pltpu.SemaphoreType.DMA,
            pltpu.SemaphoreType.DMA,
            pltpu.SemaphoreType.DMA,
            pltpu.SemaphoreType.DMA,
            pltpu.SemaphoreType.DMA,
        ],
        compiler_params=pltpu.CompilerParams(needs_layout_passes=False),
    )
    def sc_encode(idx_hbm, tab_hbm, pos2_hbm, out_hbm,
                  idx_v, bufs, outbufs,
                  gsem0, gsem1, osem0, osem1, psem0, psem1):
        gsems = (gsem0, gsem1)
        osems = (osem0, osem1)
        psems = (psem0, psem1)
        wid = lax.axis_index("s") * 2 + lax.axis_index("c")
        base = pl.multiple_of(wid * rpw, rpw)
        pltpu.sync_copy(idx_hbm.at[pl.ds(base, rpw)], idx_v)

        def gather_start(c, b):
            off = pl.multiple_of(c * _CHUNK, _CHUNK)
            pltpu.async_copy(
                tab_hbm.at[idx_v.at[pl.ds(off, _CHUNK)]], bufs.at[b], gsems[b]
            )

        def gather_wait(c, b):
            off = pl.multiple_of(c * _CHUNK, _CHUNK)
            pltpu.make_async_copy(
                tab_hbm.at[idx_v.at[pl.ds(off, _CHUNK)]], bufs.at[b], gsems[b]
            ).wait()

        def out_rows(c):
            off = pl.multiple_of(c * _CHUNK, _CHUNK)
            return out_hbm.at[pl.ds(base + off, _CHUNK)]

        def posfill_start(c, b):
            p0 = lax.rem(c * _CHUNK, seq_len)
            pltpu.async_copy(
                pos2_hbm.at[pl.ds(p0, _CHUNK)], outbufs.at[b], psems[b]
            )

        def posfill_wait(c, b):
            p0 = lax.rem(c * _CHUNK, seq_len)
            pltpu.make_async_copy(
                pos2_hbm.at[pl.ds(p0, _CHUNK)], outbufs.at[b], psems[b]
            ).wait()

        def process(c, b):
            buf = bufs.at[b]
            outb = outbufs.at[b]

            def row_body(i, carry):
                r0 = i * _UNROLL
                for rr in range(_UNROLL):
                    r = r0 + rr
                    for k in range(n_vec):
                        sl = pl.ds(k * _LANES, _LANES)
                        plsc.addupdate(outb.at[r, sl], buf[r, sl])
                return carry

            lax.fori_loop(0, _CHUNK // _UNROLL, row_body, 0)
            pltpu.sync_copy(outb, out_rows(c))

        def store_wait(c, b):
            pltpu.make_async_copy(outbufs.at[b], out_rows(c), osems[b]).wait()

        gather_start(0, 0)
        gather_start(1, 1)

        def chunk_pair(j, carry):
            for b in range(2):
                c = 2 * j + b
                posfill_start(c, b)
                gather_wait(c, b)
                posfill_wait(c, b)
                process(c, b)
                gather_start(lax.rem(c + 2, n_chunks), b)
            return carry

        lax.fori_loop(0, n_chunks // 2, chunk_pair, 0)
        gather_wait(0, 0)
        gather_wait(1, 1)

    return sc_encode


def kernel(inputs, emb_table, pos_table):
    batch, seq_len = inputs.shape
    n_vocab, d_model = emb_table.shape
    n_rows = batch * seq_len
    idx_flat = inputs.reshape(n_rows)
    tab_pad = jnp.pad(emb_table, ((0, 0), (0, d_model)))
    pos2 = jnp.concatenate([pos_table, pos_table], axis=0)
    out = _build_sc_call(n_rows, seq_len, d_model)(idx_flat, tab_pad, pos2)
    return out.reshape(batch, seq_len, d_model)


# speedup vs baseline: 1.1456x; 1.0956x over previous
import functools

import jax
import jax.numpy as jnp
from jax import lax
from jax.experimental import pallas as pl
from jax.experimental.pallas import tpu as pltpu
from jax.experimental.pallas import tpu_sc as plsc

_LANES = 16
_NUM_WORKERS = 32
_CHUNK = 128
_TBLK = 2048


def _repack_table(emb_t, n_vocab, d_model):
    n_blocks = pl.cdiv(n_vocab, _TBLK)

    def body(in_ref, out_ref):
        t = in_ref[...].T
        out_ref[...] = jnp.concatenate([t, t], axis=1)

    return pl.pallas_call(
        body,
        grid=(n_blocks,),
        in_specs=[pl.BlockSpec((d_model, _TBLK), lambda i: (0, i))],
        out_specs=pl.BlockSpec((_TBLK, 2 * d_model), lambda i: (i, 0)),
        out_shape=jax.ShapeDtypeStruct((n_vocab, 2 * d_model), jnp.float32),
    )(emb_t)


def _build_sc_call(n_rows, seq_len, d_model):
    rpw = n_rows // _NUM_WORKERS
    n_chunks = rpw // _CHUNK
    n_vec = d_model // _LANES
    mesh = plsc.VectorSubcoreMesh(
        core_axis_name="c", subcore_axis_name="s", num_cores=2, num_subcores=16
    )

    @functools.partial(
        pl.kernel,
        out_type=jax.ShapeDtypeStruct((n_rows, d_model), jnp.float32),
        mesh=mesh,
        scratch_types=[
            pltpu.VMEM((rpw,), jnp.int32),
            pltpu.VMEM((2 * seq_len, d_model), jnp.float32),
            pltpu.VMEM((2, _CHUNK, 2 * d_model), jnp.float32),
            pltpu.VMEM((_CHUNK, d_model), jnp.float32),
            pltpu.SemaphoreType.DMA,
            pltpu.SemaphoreType.DMA,
        ],
        compiler_params=pltpu.CompilerParams(needs_layout_passes=False),
    )
    def sc_encode(idx_hbm, tab_hbm, pos2_hbm, out_hbm,
                  idx_v, pos_v, bufs, outb, sem0, sem1):
        sems = (sem0, sem1)
        wid = lax.axis_index("s") * 2 + lax.axis_index("c")
        base = pl.multiple_of(wid * rpw, rpw)
        pltpu.sync_copy(idx_hbm.at[pl.ds(base, rpw)], idx_v)
        pltpu.sync_copy(pos2_hbm, pos_v)

        def gather_start(c, b):
            off = pl.multiple_of(c * _CHUNK, _CHUNK)
            pltpu.async_copy(
                tab_hbm.at[idx_v.at[pl.ds(off, _CHUNK)]], bufs.at[b], sems[b]
            )

        def gather_wait(c, b):
            off = pl.multiple_of(c * _CHUNK, _CHUNK)
            pltpu.make_async_copy(
                tab_hbm.at[idx_v.at[pl.ds(off, _CHUNK)]], bufs.at[b], sems[b]
            ).wait()

        def process(c, b):
            p0 = lax.rem(c * _CHUNK, seq_len)
            off = pl.multiple_of(c * _CHUNK, _CHUNK)
            buf = bufs.at[b]

            def row_body(r, carry):
                pr = p0 + r
                for k in range(n_vec):
                    sl = pl.ds(k * _LANES, _LANES)
                    outb[r, sl] = buf[r, sl] + pos_v[pr, sl]
                return carry

            lax.fori_loop(0, _CHUNK, row_body, 0)
            pltpu.sync_copy(outb, out_hbm.at[pl.ds(base + off, _CHUNK)])

        gather_start(0, 0)

        def chunk_pair(j, carry):
            for b in range(2):
                c = 2 * j + b
                gather_start(lax.rem(c + 1, n_chunks), 1 - b)
                gather_wait(c, b)
                process(c, b)
            return carry

        lax.fori_loop(0, n_chunks // 2, chunk_pair, 0)
        gather_wait(0, 0)

    return sc_encode


def kernel(inputs, emb_table, pos_table):
    batch, seq_len = inputs.shape
    n_vocab, d_model = emb_table.shape
    n_rows = batch * seq_len
    idx_flat = inputs.reshape(n_rows)
    tab_pad = _repack_table(emb_table.T, n_vocab, d_model)
    pos2 = jnp.concatenate([pos_table, pos_table], axis=0)
    out = _build_sc_call(n_rows, seq_len, d_model)(idx_flat, tab_pad, pos2)
    return out.reshape(batch, seq_len, d_model)


# speedup vs baseline: 1.5394x; 1.3437x over previous
import functools

import jax
import jax.numpy as jnp
from jax import lax
from jax.experimental import pallas as pl
from jax.experimental.pallas import tpu as pltpu
from jax.experimental.pallas import tpu_sc as plsc

_LANES = 16
_NUM_WORKERS = 32
_CHUNK = 128
_TBLK = 8192


def _repack_table(emb_t, n_vocab, d_model):
    n_blocks = pl.cdiv(n_vocab, _TBLK)

    def body(in_ref, out_ref):
        t = in_ref[...].T
        out_ref[...] = jnp.concatenate([t, t], axis=1)

    return pl.pallas_call(
        body,
        grid=(n_blocks,),
        in_specs=[pl.BlockSpec((d_model, _TBLK), lambda i: (0, i))],
        out_specs=pl.BlockSpec((_TBLK, 2 * d_model), lambda i: (i, 0)),
        out_shape=jax.ShapeDtypeStruct((n_vocab, 2 * d_model), jnp.float32),
    )(emb_t)


def _build_sc_call(n_rows, seq_len, d_model):
    rpw = n_rows // _NUM_WORKERS
    n_chunks = rpw // _CHUNK
    n_vec = d_model // _LANES
    mesh = plsc.VectorSubcoreMesh(
        core_axis_name="c", subcore_axis_name="s", num_cores=2, num_subcores=16
    )

    @functools.partial(
        pl.kernel,
        out_type=jax.ShapeDtypeStruct((n_rows, d_model), jnp.float32),
        mesh=mesh,
        scratch_types=[
            pltpu.VMEM((rpw,), jnp.int32),
            pltpu.VMEM((2 * seq_len, d_model), jnp.float32),
            pltpu.VMEM((2, _CHUNK, 2 * d_model), jnp.float32),
            pltpu.VMEM((_CHUNK, d_model), jnp.float32),
            pltpu.SemaphoreType.DMA,
            pltpu.SemaphoreType.DMA,
        ],
        compiler_params=pltpu.CompilerParams(needs_layout_passes=False),
    )
    def sc_encode(idx_hbm, tab_hbm, pos2_hbm, out_hbm,
                  idx_v, pos_v, bufs, outb, sem0, sem1):
        sems = (sem0, sem1)
        wid = lax.axis_index("s") * 2 + lax.axis_index("c")
        base = pl.multiple_of(wid * rpw, rpw)
        pltpu.sync_copy(idx_hbm.at[pl.ds(base, rpw)], idx_v)
        pltpu.sync_copy(pos2_hbm, pos_v)

        def gather_start(c, b):
            off = pl.multiple_of(c * _CHUNK, _CHUNK)
            pltpu.async_copy(
                tab_hbm.at[idx_v.at[pl.ds(off, _CHUNK)]], bufs.at[b], sems[b]
            )

        def gather_wait(c, b):
            off = pl.multiple_of(c * _CHUNK, _CHUNK)
            pltpu.make_async_copy(
                tab_hbm.at[idx_v.at[pl.ds(off, _CHUNK)]], bufs.at[b], sems[b]
            ).wait()

        def process(c, b):
            p0 = lax.rem(c * _CHUNK, seq_len)
            off = pl.multiple_of(c * _CHUNK, _CHUNK)
            buf = bufs.at[b]

            def row_body(r, carry):
                pr = p0 + r
                for k in range(n_vec):
                    sl = pl.ds(k * _LANES, _LANES)
                    outb[r, sl] = buf[r, sl] + pos_v[pr, sl]
                return carry

            lax.fori_loop(0, _CHUNK, row_body, 0)
            pltpu.sync_copy(outb, out_hbm.at[pl.ds(base + off, _CHUNK)])

        gather_start(0, 0)

        def chunk_pair(j, carry):
            for b in range(2):
                c = 2 * j + b
                gather_start(lax.rem(c + 1, n_chunks), 1 - b)
                gather_wait(c, b)
                process(c, b)
            return carry

        lax.fori_loop(0, n_chunks // 2, chunk_pair, 0)
        gather_wait(0, 0)

    return sc_encode


def kernel(inputs, emb_table, pos_table):
    batch, seq_len = inputs.shape
    n_vocab, d_model = emb_table.shape
    n_rows = batch * seq_len
    idx_flat = inputs.reshape(n_rows)
    tab_pad = _repack_table(emb_table.T, n_vocab, d_model)
    pos2 = jnp.concatenate([pos_table, pos_table], axis=0)
    out = _build_sc_call(n_rows, seq_len, d_model)(idx_flat, tab_pad, pos2)
    return out.reshape(batch, seq_len, d_model)


# speedup vs baseline: 1.6441x; 1.0680x over previous
import functools

import jax
import jax.numpy as jnp
from jax import lax
from jax.experimental import pallas as pl
from jax.experimental.pallas import tpu as pltpu
from jax.experimental.pallas import tpu_sc as plsc

_LANES = 16
_NUM_WORKERS = 32
_CHUNK = 128
_TBLK = 16384


def _repack_table(emb_t, n_vocab, d_model):
    n_blocks = pl.cdiv(n_vocab, _TBLK)

    def body(in_ref, out_ref):
        t = in_ref[...].T
        out_ref[...] = jnp.concatenate([t, t], axis=1)

    return pl.pallas_call(
        body,
        grid=(n_blocks,),
        in_specs=[pl.BlockSpec((d_model, _TBLK), lambda i: (0, i))],
        out_specs=pl.BlockSpec((_TBLK, 2 * d_model), lambda i: (i, 0)),
        out_shape=jax.ShapeDtypeStruct((n_vocab, 2 * d_model), jnp.float32),
    )(emb_t)


def _build_sc_call(n_rows, seq_len, d_model):
    rpw = n_rows // _NUM_WORKERS
    n_chunks = rpw // _CHUNK
    n_vec = d_model // _LANES
    mesh = plsc.VectorSubcoreMesh(
        core_axis_name="c", subcore_axis_name="s", num_cores=2, num_subcores=16
    )

    @functools.partial(
        pl.kernel,
        out_type=jax.ShapeDtypeStruct((n_rows, d_model), jnp.float32),
        mesh=mesh,
        scratch_types=[
            pltpu.VMEM((rpw,), jnp.int32),
            pltpu.VMEM((2 * seq_len, d_model), jnp.float32),
            pltpu.VMEM((2, _CHUNK, 2 * d_model), jnp.float32),
            pltpu.VMEM((_CHUNK, d_model), jnp.float32),
            pltpu.SemaphoreType.DMA,
            pltpu.SemaphoreType.DMA,
        ],
        compiler_params=pltpu.CompilerParams(needs_layout_passes=False),
    )
    def sc_encode(idx_hbm, tab_hbm, pos2_hbm, out_hbm,
                  idx_v, pos_v, bufs, outb, sem0, sem1):
        sems = (sem0, sem1)
        wid = lax.axis_index("s") * 2 + lax.axis_index("c")
        base = pl.multiple_of(wid * rpw, rpw)
        pltpu.sync_copy(idx_hbm.at[pl.ds(base, rpw)], idx_v)
        pltpu.sync_copy(pos2_hbm, pos_v)

        def gather_start(c, b):
            off = pl.multiple_of(c * _CHUNK, _CHUNK)
            pltpu.async_copy(
                tab_hbm.at[idx_v.at[pl.ds(off, _CHUNK)]], bufs.at[b], sems[b]
            )

        def gather_wait(c, b):
            off = pl.multiple_of(c * _CHUNK, _CHUNK)
            pltpu.make_async_copy(
                tab_hbm.at[idx_v.at[pl.ds(off, _CHUNK)]], bufs.at[b], sems[b]
            ).wait()

        def process(c, b):
            p0 = lax.rem(c * _CHUNK, seq_len)
            off = pl.multiple_of(c * _CHUNK, _CHUNK)
            buf = bufs.at[b]

            def row_body(r, carry):
                pr = p0 + r
                for k in range(n_vec):
                    sl = pl.ds(k * _LANES, _LANES)
                    outb[r, sl] = buf[r, sl] + pos_v[pr, sl]
                return carry

            lax.fori_loop(0, _CHUNK, row_body, 0)
            pltpu.sync_copy(outb, out_hbm.at[pl.ds(base + off, _CHUNK)])

        gather_start(0, 0)

        def chunk_pair(j, carry):
            for b in range(2):
                c = 2 * j + b
                gather_start(lax.rem(c + 1, n_chunks), 1 - b)
                gather_wait(c, b)
                process(c, b)
            return carry

        lax.fori_loop(0, n_chunks // 2, chunk_pair, 0)
        gather_wait(0, 0)

    return sc_encode


def kernel(inputs, emb_table, pos_table):
    batch, seq_len = inputs.shape
    n_vocab, d_model = emb_table.shape
    n_rows = batch * seq_len
    idx_flat = inputs.reshape(n_rows)
    tab_pad = _repack_table(emb_table.T, n_vocab, d_model)
    pos2 = jnp.concatenate([pos_table, pos_table], axis=0)
    out = _build_sc_call(n_rows, seq_len, d_model)(idx_flat, tab_pad, pos2)
    return out.reshape(batch, seq_len, d_model)


# speedup vs baseline: 1.6757x; 1.0192x over previous
import functools

import jax
import jax.numpy as jnp
from jax import lax
from jax.experimental import pallas as pl
from jax.experimental.pallas import tpu as pltpu
from jax.experimental.pallas import tpu_sc as plsc

_LANES = 16
_NUM_WORKERS = 32
_CHUNK = 128
_TBLK = 24576


def _repack_table(emb_t, n_vocab, d_model):
    n_blocks = pl.cdiv(n_vocab, _TBLK)

    def body(in_ref, out_ref):
        t = in_ref[...].T
        out_ref[...] = jnp.concatenate([t, t], axis=1)

    return pl.pallas_call(
        body,
        grid=(n_blocks,),
        in_specs=[pl.BlockSpec((d_model, _TBLK), lambda i: (0, i))],
        out_specs=pl.BlockSpec((_TBLK, 2 * d_model), lambda i: (i, 0)),
        out_shape=jax.ShapeDtypeStruct((n_vocab, 2 * d_model), jnp.float32),
    )(emb_t)


def _build_sc_call(n_rows, seq_len, d_model):
    rpw = n_rows // _NUM_WORKERS
    n_chunks = rpw // _CHUNK
    n_vec = d_model // _LANES
    mesh = plsc.VectorSubcoreMesh(
        core_axis_name="c", subcore_axis_name="s", num_cores=2, num_subcores=16
    )

    @functools.partial(
        pl.kernel,
        out_type=jax.ShapeDtypeStruct((n_rows, d_model), jnp.float32),
        mesh=mesh,
        scratch_types=[
            pltpu.VMEM((rpw,), jnp.int32),
            pltpu.VMEM((2 * seq_len, d_model), jnp.float32),
            pltpu.VMEM((2, _CHUNK, 2 * d_model), jnp.float32),
            pltpu.VMEM((_CHUNK, d_model), jnp.float32),
            pltpu.SemaphoreType.DMA,
            pltpu.SemaphoreType.DMA,
        ],
        compiler_params=pltpu.CompilerParams(needs_layout_passes=False),
    )
    def sc_encode(idx_hbm, tab_hbm, pos2_hbm, out_hbm,
                  idx_v, pos_v, bufs, outb, sem0, sem1):
        sems = (sem0, sem1)
        wid = lax.axis_index("s") * 2 + lax.axis_index("c")
        base = pl.multiple_of(wid * rpw, rpw)
        pltpu.sync_copy(idx_hbm.at[pl.ds(base, rpw)], idx_v)
        pltpu.sync_copy(pos2_hbm, pos_v)

        def gather_start(c, b):
            off = pl.multiple_of(c * _CHUNK, _CHUNK)
            pltpu.async_copy(
                tab_hbm.at[idx_v.at[pl.ds(off, _CHUNK)]], bufs.at[b], sems[b]
            )

        def gather_wait(c, b):
            off = pl.multiple_of(c * _CHUNK, _CHUNK)
            pltpu.make_async_copy(
                tab_hbm.at[idx_v.at[pl.ds(off, _CHUNK)]], bufs.at[b], sems[b]
            ).wait()

        def process(c, b):
            p0 = lax.rem(c * _CHUNK, seq_len)
            off = pl.multiple_of(c * _CHUNK, _CHUNK)
            buf = bufs.at[b]

            def row_body(r, carry):
                pr = p0 + r
                for k in range(n_vec):
                    sl = pl.ds(k * _LANES, _LANES)
                    outb[r, sl] = buf[r, sl] + pos_v[pr, sl]
                return carry

            lax.fori_loop(0, _CHUNK, row_body, 0)
            pltpu.sync_copy(outb, out_hbm.at[pl.ds(base + off, _CHUNK)])

        gather_start(0, 0)

        def chunk_pair(j, carry):
            for b in range(2):
                c = 2 * j + b
                gather_start(lax.rem(c + 1, n_chunks), 1 - b)
                gather_wait(c, b)
                process(c, b)
            return carry

        lax.fori_loop(0, n_chunks // 2, chunk_pair, 0)
        gather_wait(0, 0)

    return sc_encode


def kernel(inputs, emb_table, pos_table):
    batch, seq_len = inputs.shape
    n_vocab, d_model = emb_table.shape
    n_rows = batch * seq_len
    idx_flat = inputs.reshape(n_rows)
    tab_pad = _repack_table(emb_table.T, n_vocab, d_model)
    pos2 = jnp.concatenate([pos_table, pos_table], axis=0)
    out = _build_sc_call(n_rows, seq_len, d_model)(idx_flat, tab_pad, pos2)
    return out.reshape(batch, seq_len, d_model)


# speedup vs baseline: 1.6830x; 1.0044x over previous
import functools

import jax
import jax.numpy as jnp
from jax import lax
from jax.experimental import pallas as pl
from jax.experimental.pallas import tpu as pltpu
from jax.experimental.pallas import tpu_sc as plsc

_LANES = 16
_NUM_WORKERS = 32
_CHUNK = 128
_TBLK = 24576


def _repack_table(emb_t, n_vocab, d_model):
    n_blocks = pl.cdiv(n_vocab, _TBLK)

    def body(in_ref, out_ref):
        t = in_ref[...].T
        out_ref[...] = jnp.concatenate([t, t], axis=1)

    return pl.pallas_call(
        body,
        grid=(n_blocks,),
        in_specs=[pl.BlockSpec((d_model, _TBLK), lambda i: (0, i))],
        out_specs=pl.BlockSpec((_TBLK, 2 * d_model), lambda i: (i, 0)),
        out_shape=jax.ShapeDtypeStruct((n_vocab, 2 * d_model), jnp.float32),
    )(emb_t)


def _build_sc_call(n_rows, seq_len, d_model):
    rpw = n_rows // _NUM_WORKERS
    n_chunks = rpw // _CHUNK
    n_vec = d_model // _LANES
    mesh = plsc.VectorSubcoreMesh(
        core_axis_name="c", subcore_axis_name="s", num_cores=2, num_subcores=16
    )

    @functools.partial(
        pl.kernel,
        out_type=jax.ShapeDtypeStruct((n_rows, d_model), jnp.float32),
        mesh=mesh,
        scratch_types=[
            pltpu.VMEM((rpw,), jnp.int32),
            pltpu.VMEM((2 * seq_len, d_model), jnp.float32),
            pltpu.VMEM((2, _CHUNK, 2 * d_model), jnp.float32),
            pltpu.VMEM((_CHUNK, d_model), jnp.float32),
            pltpu.SemaphoreType.DMA,
            pltpu.SemaphoreType.DMA,
        ],
        compiler_params=pltpu.CompilerParams(needs_layout_passes=False),
    )
    def sc_encode(idx_hbm, tab_hbm, pos2_hbm, out_hbm,
                  idx_v, pos_v, bufs, outb, sem0, sem1):
        sems = (sem0, sem1)
        wid = lax.axis_index("s") * 2 + lax.axis_index("c")
        base = pl.multiple_of(wid * rpw, rpw)
        pltpu.sync_copy(idx_hbm.at[pl.ds(base, rpw)], idx_v)
        pltpu.sync_copy(pos2_hbm, pos_v)

        def gather_start(c, b):
            off = pl.multiple_of(c * _CHUNK, _CHUNK)
            pltpu.async_copy(
                tab_hbm.at[idx_v.at[pl.ds(off, _CHUNK)]], bufs.at[b], sems[b]
            )

        def gather_wait(c, b):
            off = pl.multiple_of(c * _CHUNK, _CHUNK)
            pltpu.make_async_copy(
                tab_hbm.at[idx_v.at[pl.ds(off, _CHUNK)]], bufs.at[b], sems[b]
            ).wait()

        def process(c, b):
            p0 = lax.rem(c * _CHUNK, seq_len)
            off = pl.multiple_of(c * _CHUNK, _CHUNK)
            buf = bufs.at[b]

            def row_body(i, carry):
                r0 = i * 4
                for rr in range(4):
                    r = r0 + rr
                    pr = p0 + r
                    for k in range(n_vec):
                        sl = pl.ds(k * _LANES, _LANES)
                        outb[r, sl] = buf[r, sl] + pos_v[pr, sl]
                return carry

            lax.fori_loop(0, _CHUNK // 4, row_body, 0)
            pltpu.sync_copy(outb, out_hbm.at[pl.ds(base + off, _CHUNK)])

        gather_start(0, 0)

        def chunk_pair(j, carry):
            for b in range(2):
                c = 2 * j + b
                gather_start(lax.rem(c + 1, n_chunks), 1 - b)
                gather_wait(c, b)
                process(c, b)
            return carry

        lax.fori_loop(0, n_chunks // 2, chunk_pair, 0)
        gather_wait(0, 0)

    return sc_encode


def kernel(inputs, emb_table, pos_table):
    batch, seq_len = inputs.shape
    n_vocab, d_model = emb_table.shape
    n_rows = batch * seq_len
    idx_flat = inputs.reshape(n_rows)
    tab_pad = _repack_table(emb_table.T, n_vocab, d_model)
    pos2 = jnp.concatenate([pos_table, pos_table], axis=0)
    out = _build_sc_call(n_rows, seq_len, d_model)(idx_flat, tab_pad, pos2)
    return out.reshape(batch, seq_len, d_model)


# speedup vs baseline: 1.7940x; 1.0659x over previous
import functools

import jax
import jax.numpy as jnp
from jax import lax
from jax.experimental import pallas as pl
from jax.experimental.pallas import tpu as pltpu
from jax.experimental.pallas import tpu_sc as plsc

_LANES = 16
_NUM_WORKERS = 32
_CHUNK = 128
_TBLK = 24576


def _repack_table(emb_t, n_vocab, d_model):
    n_blocks = pl.cdiv(n_vocab, _TBLK)

    def body(in_ref, out_ref):
        t = in_ref[...].T
        out_ref[...] = jnp.concatenate([t, t], axis=1)

    return pl.pallas_call(
        body,
        grid=(n_blocks,),
        in_specs=[pl.BlockSpec((d_model, _TBLK), lambda i: (0, i))],
        out_specs=pl.BlockSpec((_TBLK, 2 * d_model), lambda i: (i, 0)),
        out_shape=jax.ShapeDtypeStruct((n_vocab, 2 * d_model), jnp.float32),
    )(emb_t)


def _build_sc_call(n_rows, seq_len, d_model):
    rpw = n_rows // _NUM_WORKERS
    n_chunks = rpw // _CHUNK
    n_vec = d_model // _LANES
    mesh = plsc.VectorSubcoreMesh(
        core_axis_name="c", subcore_axis_name="s", num_cores=2, num_subcores=16
    )

    @functools.partial(
        pl.kernel,
        out_type=jax.ShapeDtypeStruct((n_rows, d_model), jnp.float32),
        mesh=mesh,
        scratch_types=[
            pltpu.VMEM((rpw,), jnp.int32),
            pltpu.VMEM((2 * seq_len, d_model), jnp.float32),
            pltpu.VMEM((2, _CHUNK, 2 * d_model), jnp.float32),
            pltpu.VMEM((2, _CHUNK, d_model), jnp.float32),
            pltpu.SemaphoreType.DMA,
            pltpu.SemaphoreType.DMA,
            pltpu.SemaphoreType.DMA,
            pltpu.SemaphoreType.DMA,
        ],
        compiler_params=pltpu.CompilerParams(needs_layout_passes=False),
    )
    def sc_encode(idx_hbm, tab_hbm, pos2_hbm, out_hbm,
                  idx_v, pos_v, bufs, outbufs, sem0, sem1, osem0, osem1):
        sems = (sem0, sem1)
        osems = (osem0, osem1)
        wid = lax.axis_index("s") * 2 + lax.axis_index("c")
        base = pl.multiple_of(wid * rpw, rpw)
        pltpu.sync_copy(idx_hbm.at[pl.ds(base, rpw)], idx_v)
        pltpu.sync_copy(pos2_hbm, pos_v)

        def gather_start(c, b):
            off = pl.multiple_of(c * _CHUNK, _CHUNK)
            pltpu.async_copy(
                tab_hbm.at[idx_v.at[pl.ds(off, _CHUNK)]], bufs.at[b], sems[b]
            )

        def gather_wait(c, b):
            off = pl.multiple_of(c * _CHUNK, _CHUNK)
            pltpu.make_async_copy(
                tab_hbm.at[idx_v.at[pl.ds(off, _CHUNK)]], bufs.at[b], sems[b]
            ).wait()

        def process(c, b):
            p0 = lax.rem(c * _CHUNK, seq_len)
            off = pl.multiple_of(c * _CHUNK, _CHUNK)
            buf = bufs.at[b]
            outb = outbufs.at[b]

            def row_body(i, carry):
                r0 = i * 4
                for rr in range(4):
                    r = r0 + rr
                    pr = p0 + r
                    for k in range(n_vec):
                        sl = pl.ds(k * _LANES, _LANES)
                        outb[r, sl] = buf[r, sl] + pos_v[pr, sl]
                return carry

            lax.fori_loop(0, _CHUNK // 4, row_body, 0)
            pltpu.async_copy(
                outb, out_hbm.at[pl.ds(base + off, _CHUNK)], osems[b]
            )

        def store_wait(c, b):
            off = pl.multiple_of(c * _CHUNK, _CHUNK)
            pltpu.make_async_copy(
                outbufs.at[b], out_hbm.at[pl.ds(base + off, _CHUNK)], osems[b]
            ).wait()

        gather_start(0, 0)

        def chunk_pair(j, carry):
            for b in range(2):
                c = 2 * j + b
                gather_start(lax.rem(c + 1, n_chunks), 1 - b)
                gather_wait(c, b)
                @pl.when(j > 0)
                def _():
                    store_wait(c - 2, b)
                process(c, b)
            return carry

        lax.fori_loop(0, n_chunks // 2, chunk_pair, 0)
        store_wait(n_chunks - 2, 0)
        store_wait(n_chunks - 1, 1)
        gather_wait(0, 0)

    return sc_encode


def kernel(inputs, emb_table, pos_table):
    batch, seq_len = inputs.shape
    n_vocab, d_model = emb_table.shape
    n_rows = batch * seq_len
    idx_flat = inputs.reshape(n_rows)
    tab_pad = _repack_table(emb_table.T, n_vocab, d_model)
    pos2 = jnp.concatenate([pos_table, pos_table], axis=0)
    out = _build_sc_call(n_rows, seq_len, d_model)(idx_flat, tab_pad, pos2)
    return out.reshape(batch, seq_len, d_model)


# speedup vs baseline: 1.7976x; 1.0020x over previous
import functools

import jax
import jax.numpy as jnp
from jax import lax
from jax.experimental import pallas as pl
from jax.experimental.pallas import tpu as pltpu
from jax.experimental.pallas import tpu_sc as plsc

_LANES = 16
_NUM_WORKERS = 32
_CHUNK = 128
_TBLK = 24576


def _repack_table(emb_t, n_vocab, d_model):
    n_blocks = pl.cdiv(n_vocab, _TBLK)

    def body(in_ref, out_ref):
        t = in_ref[...].T
        out_ref[...] = jnp.concatenate([t, t], axis=1)

    return pl.pallas_call(
        body,
        grid=(n_blocks,),
        in_specs=[pl.BlockSpec((d_model, _TBLK), lambda i: (0, i))],
        out_specs=pl.BlockSpec((_TBLK, 2 * d_model), lambda i: (i, 0)),
        out_shape=jax.ShapeDtypeStruct((n_vocab, 2 * d_model), jnp.float32),
    )(emb_t)


def _build_sc_call(n_rows, seq_len, d_model):
    rpw = n_rows // _NUM_WORKERS
    n_chunks = rpw // _CHUNK
    n_vec = d_model // _LANES
    mesh = plsc.VectorSubcoreMesh(
        core_axis_name="c", subcore_axis_name="s", num_cores=2, num_subcores=16
    )

    @functools.partial(
        pl.kernel,
        out_type=jax.ShapeDtypeStruct((n_rows, d_model), jnp.float32),
        mesh=mesh,
        scratch_types=[
            pltpu.VMEM((rpw,), jnp.int32),
            pltpu.VMEM((2 * seq_len, d_model), jnp.float32),
            pltpu.VMEM((2, _CHUNK, 2 * d_model), jnp.float32),
            pltpu.VMEM((2, _CHUNK, d_model), jnp.float32),
            pltpu.SemaphoreType.DMA,
            pltpu.SemaphoreType.DMA,
            pltpu.SemaphoreType.DMA,
            pltpu.SemaphoreType.DMA,
        ],
        compiler_params=pltpu.CompilerParams(needs_layout_passes=False),
    )
    def sc_encode(idx_hbm, tab_hbm, pos2_hbm, out_hbm,
                  idx_v, pos_v, bufs, outbufs, sem0, sem1, osem0, osem1):
        sems = (sem0, sem1)
        osems = (osem0, osem1)
        wid = lax.axis_index("s") * 2 + lax.axis_index("c")
        base = pl.multiple_of(wid * rpw, rpw)
        pltpu.sync_copy(idx_hbm.at[pl.ds(base, rpw)], idx_v)
        pltpu.sync_copy(pos2_hbm, pos_v)

        def gather_start(c, b):
            off = pl.multiple_of(c * _CHUNK, _CHUNK)
            pltpu.async_copy(
                tab_hbm.at[idx_v.at[pl.ds(off, _CHUNK)]], bufs.at[b], sems[b]
            )

        def gather_wait(c, b):
            off = pl.multiple_of(c * _CHUNK, _CHUNK)
            pltpu.make_async_copy(
                tab_hbm.at[idx_v.at[pl.ds(off, _CHUNK)]], bufs.at[b], sems[b]
            ).wait()

        def process(c, b):
            p0 = lax.rem(c * _CHUNK, seq_len)
            off = pl.multiple_of(c * _CHUNK, _CHUNK)
            buf = bufs.at[b]
            outb = outbufs.at[b]

            def row_body(i, carry):
                r0 = i * 8
                for rr in range(8):
                    r = r0 + rr
                    pr = p0 + r
                    for k in range(n_vec):
                        sl = pl.ds(k * _LANES, _LANES)
                        outb[r, sl] = buf[r, sl] + pos_v[pr, sl]
                return carry

            lax.fori_loop(0, _CHUNK // 8, row_body, 0)
            pltpu.async_copy(
                outb, out_hbm.at[pl.ds(base + off, _CHUNK)], osems[b]
            )

        def store_wait(c, b):
            off = pl.multiple_of(c * _CHUNK, _CHUNK)
            pltpu.make_async_copy(
                outbufs.at[b], out_hbm.at[pl.ds(base + off, _CHUNK)], osems[b]
            ).wait()

        gather_start(0, 0)

        def chunk_pair(j, carry):
            for b in range(2):
                c = 2 * j + b
                gather_start(lax.rem(c + 1, n_chunks), 1 - b)
                gather_wait(c, b)
                @pl.when(j > 0)
                def _():
                    store_wait(c - 2, b)
                process(c, b)
            return carry

        lax.fori_loop(0, n_chunks // 2, chunk_pair, 0)
        store_wait(n_chunks - 2, 0)
        store_wait(n_chunks - 1, 1)
        gather_wait(0, 0)

    return sc_encode


def kernel(inputs, emb_table, pos_table):
    batch, seq_len = inputs.shape
    n_vocab, d_model = emb_table.shape
    n_rows = batch * seq_len
    idx_flat = inputs.reshape(n_rows)
    tab_pad = _repack_table(emb_table.T, n_vocab, d_model)
    pos2 = jnp.concatenate([pos_table, pos_table], axis=0)
    out = _build_sc_call(n_rows, seq_len, d_model)(idx_flat, tab_pad, pos2)
    return out.reshape(batch, seq_len, d_model)
